# baseline (speedup 1.0000x reference)
import functools

import jax
import jax.numpy as jnp
from jax import lax
from jax.experimental import pallas as pl
from jax.experimental.pallas import tpu as pltpu
from jax.experimental.pallas import tpu_sc as plsc

N_NODES = 10000
N_EDGES = 320000
D_HID = 128
N_GRAPHS = 128
D_OUT = 10

NC = 2
NS = 16
NW = NC * NS
ECHUNK = 128
E_PAD = 327680
EROWS = E_PAD // ECHUNK
ROWS_PER_W = EROWS // NW
N_PAD = 10240
ROWS_PER_TILE = N_PAD // NS
ROW_COPY = 128

_sc_mesh = plsc.VectorSubcoreMesh(core_axis_name="c", subcore_axis_name="s",
                                  num_cores=NC, num_subcores=NS)


def _fill(ref, val):
    r, c = ref.shape

    def body(i, _):
        for j in range(c // 16):
            ref[i, pl.ds(j * 16, 16)] = jnp.full((16,), val, jnp.float32)
        return 0

    lax.fori_loop(0, r, body, 0)


DEG_WIN = 8


@functools.partial(
    pl.kernel,
    out_type=jax.ShapeDtypeStruct((NC, N_PAD, D_HID), jnp.float32),
    mesh=_sc_mesh,
    scratch_types=[
        pltpu.VMEM((ROWS_PER_W, ECHUNK), jnp.int32),
        pltpu.VMEM((ECHUNK, D_HID), jnp.float32),
        pltpu.VMEM_SHARED((N_PAD, D_HID), jnp.float32),
        pltpu.SemaphoreType.DMA,
    ],
)
def _sc_degree(dst_hbm, out_hbm, dsts_v, ones_v, acc_sh, sem):
    c = lax.axis_index("c")
    s = lax.axis_index("s")
    wid = s * NC + c
    erow0 = wid * ROWS_PER_W

    _fill(ones_v, 0.0)
    base = s * ROWS_PER_TILE
    for k in range(ROWS_PER_TILE // ROW_COPY):
        pltpu.sync_copy(ones_v, acc_sh.at[pl.ds(base + k * ROW_COPY, ROW_COPY)])
    _fill(ones_v, 1.0)
    pltpu.sync_copy(dst_hbm.at[pl.ds(erow0, ROWS_PER_W)], dsts_v)
    plsc.subcore_barrier()

    def body(k, _):
        pltpu.async_copy(ones_v, acc_sh.at[dsts_v.at[k]], sem, add=True)

        @pl.when(k >= DEG_WIN)
        def _():
            pltpu.make_async_copy(ones_v, acc_sh.at[dsts_v.at[0]], sem).wait()

        return 0

    lax.fori_loop(0, ROWS_PER_W, body, 0)
    for _ in range(DEG_WIN):
        pltpu.make_async_copy(ones_v, acc_sh.at[dsts_v.at[0]], sem).wait()
    plsc.subcore_barrier()

    for k in range(ROWS_PER_TILE // ROW_COPY):
        r0 = base + k * ROW_COPY
        pltpu.sync_copy(acc_sh.at[pl.ds(r0, ROW_COPY)], ones_v)
        pltpu.sync_copy(ones_v, out_hbm.at[c, pl.ds(r0, ROW_COPY)])


@functools.partial(
    pl.kernel,
    out_type=jax.ShapeDtypeStruct((NC, N_PAD, D_HID), jnp.float32),
    mesh=_sc_mesh,
    scratch_types=[
        pltpu.VMEM((ROWS_PER_W // 2, ECHUNK), jnp.int32),
        pltpu.VMEM((ROWS_PER_W // 2, ECHUNK), jnp.int32),
        pltpu.VMEM((2, ECHUNK, D_HID), jnp.float32),
        pltpu.VMEM_SHARED((N_PAD, D_HID), jnp.float32),
        pltpu.SemaphoreType.DMA,
        pltpu.SemaphoreType.DMA,
    ],
)
def _sc_message(y_hbm, src_hbm, dst_hbm, out_hbm, srcs_v, dsts_v, rows_v,
                acc_sh, gsem0, gsem1):
    c = lax.axis_index("c")
    s = lax.axis_index("s")
    wid = s * NC + c
    erow0 = wid * ROWS_PER_W

    _fill(rows_v.at[0], 0.0)
    base = s * ROWS_PER_TILE
    for k in range(ROWS_PER_TILE // ROW_COPY):
        pltpu.sync_copy(rows_v.at[0], acc_sh.at[pl.ds(base + k * ROW_COPY, ROW_COPY)])
    plsc.subcore_barrier()

    half = ROWS_PER_W // 2
    for h in range(2):
        pltpu.sync_copy(src_hbm.at[pl.ds(erow0 + h * half, half)], srcs_v)
        pltpu.sync_copy(dst_hbm.at[pl.ds(erow0 + h * half, half)], dsts_v)
        pltpu.async_copy(y_hbm.at[srcs_v.at[0]], rows_v.at[0], gsem0)

        def body(i, _):
            k0 = 2 * i
            k1 = k0 + 1
            pltpu.make_async_copy(y_hbm.at[srcs_v.at[k0]], rows_v.at[0], gsem0).wait()
            pltpu.async_copy(y_hbm.at[srcs_v.at[k1]], rows_v.at[1], gsem1)
            pltpu.sync_copy(rows_v.at[0], acc_sh.at[dsts_v.at[k0]], add=True)
            pltpu.make_async_copy(y_hbm.at[srcs_v.at[k1]], rows_v.at[1], gsem1).wait()

            @pl.when(i < half // 2 - 1)
            def _():
                pltpu.async_copy(y_hbm.at[srcs_v.at[k0 + 2]], rows_v.at[0], gsem0)

            pltpu.sync_copy(rows_v.at[1], acc_sh.at[dsts_v.at[k1]], add=True)
            return 0

        lax.fori_loop(0, half // 2, body, 0)
    plsc.subcore_barrier()

    for k in range(ROWS_PER_TILE // ROW_COPY):
        r0 = base + k * ROW_COPY
        pltpu.sync_copy(acc_sh.at[pl.ds(r0, ROW_COPY)], rows_v.at[0])
        pltpu.sync_copy(rows_v.at[0], out_hbm.at[c, pl.ds(r0, ROW_COPY)])


BN = 2048


def _dinv_block(d0_ref, d1_ref):
    deg = d0_ref[:, 0:1] + d1_ref[:, 0:1] + 1.0
    return lax.rsqrt(deg)


def _tc_layer1_body(x_ref, w_ref, d0_ref, d1_ref, y_ref):
    dinv = _dinv_block(d0_ref, d1_ref)
    y_ref[...] = dinv * jnp.dot(x_ref[...], w_ref[...], preferred_element_type=jnp.float32)


def _tc_layer_body(a0_ref, a1_ref, y_ref, d0_ref, d1_ref, b_ref, w_ref, out_ref):
    dinv = _dinv_block(d0_ref, d1_ref)
    h = dinv * (a0_ref[...] + a1_ref[...] + y_ref[...]) + b_ref[...]
    h = jnp.maximum(h, 0.0)
    out_ref[...] = dinv * jnp.dot(h, w_ref[...], preferred_element_type=jnp.float32)


def _tc_final_body(a0_ref, a1_ref, y_ref, d0_ref, d1_ref, b_ref, batch_ref,
                   lw_ref, lb_ref, out_ref, pool_acc, cnt_acc):
    j = pl.program_id(0)
    dinv = _dinv_block(d0_ref, d1_ref)
    h = dinv * (a0_ref[...] + a1_ref[...] + y_ref[...]) + b_ref[...]

    bblk = batch_ref[0, :]
    onehot = (lax.broadcasted_iota(jnp.int32, (N_GRAPHS, BN), 0)
              == bblk[None, :]).astype(jnp.float32)

    @pl.when(j == 0)
    def _():
        pool_acc[...] = jnp.zeros_like(pool_acc)
        cnt_acc[...] = jnp.zeros_like(cnt_acc)

    pool_acc[...] += jnp.dot(onehot, h, preferred_element_type=jnp.float32)
    cnt_acc[...] += jnp.sum(onehot, axis=1, keepdims=True)

    @pl.when(j == pl.num_programs(0) - 1)
    def _():
        pooled = pool_acc[...] / jnp.maximum(cnt_acc[...], 1.0)
        out_ref[...] = (jnp.dot(pooled, lw_ref[...], preferred_element_type=jnp.float32)
                        + lb_ref[...])


_GRID = N_PAD // BN

_node_spec = pl.BlockSpec((BN, D_HID), lambda j: (j, 0))
_deg_spec = pl.BlockSpec((BN, D_HID), lambda j: (j, 0))
_full_spec = pl.BlockSpec((D_HID, D_HID), lambda j: (0, 0))
_bias_spec = pl.BlockSpec((1, D_HID), lambda j: (0, 0))


def _tc_layer1(x, W, d0, d1):
    return pl.pallas_call(
        _tc_layer1_body,
        grid=(_GRID,),
        in_specs=[_node_spec, _full_spec, _deg_spec, _deg_spec],
        out_specs=_node_spec,
        out_shape=jax.ShapeDtypeStruct((N_PAD, D_HID), jnp.float32),
    )(x, W, d0, d1)


def _tc_layer(a0, a1, y, d0, d1, b, W):
    return pl.pallas_call(
        _tc_layer_body,
        grid=(_GRID,),
        in_specs=[_node_spec, _node_spec, _node_spec, _deg_spec, _deg_spec,
                  _bias_spec, _full_spec],
        out_specs=_node_spec,
        out_shape=jax.ShapeDtypeStruct((N_PAD, D_HID), jnp.float32),
    )(a0, a1, y, d0, d1, b, W)


def _tc_final(a0, a1, y, d0, d1, b, batch2, lin_W, lin_b):
    return pl.pallas_call(
        _tc_final_body,
        grid=(_GRID,),
        in_specs=[_node_spec, _node_spec, _node_spec, _deg_spec, _deg_spec,
                  _bias_spec,
                  pl.BlockSpec((1, BN), lambda j: (0, j)),
                  pl.BlockSpec((D_HID, D_OUT), lambda j: (0, 0)),
                  pl.BlockSpec((1, D_OUT), lambda j: (0, 0))],
        out_specs=pl.BlockSpec((N_GRAPHS, D_OUT), lambda j: (0, 0)),
        out_shape=jax.ShapeDtypeStruct((N_GRAPHS, D_OUT), jnp.float32),
        scratch_shapes=[pltpu.VMEM((N_GRAPHS, D_HID), jnp.float32),
                        pltpu.VMEM((N_GRAPHS, 1), jnp.float32)],
    )(a0, a1, y, d0, d1, b, batch2, lin_W, lin_b)


def kernel(x, edge_index, batch, W1, b1, W2, b2, W3, b3, lin_W, lin_b):
    x = jnp.pad(x.astype(jnp.float32), ((0, N_PAD - N_NODES), (0, 0)))
    src = edge_index[0].astype(jnp.int32)
    dst = edge_index[1].astype(jnp.int32)
    src2 = jnp.pad(src, (0, E_PAD - N_EDGES)).reshape(EROWS, ECHUNK)
    dst2 = jnp.pad(dst, (0, E_PAD - N_EDGES),
                   constant_values=N_PAD - 1).reshape(EROWS, ECHUNK)
    batch2 = jnp.pad(batch.astype(jnp.int32), (0, N_PAD - N_NODES),
                     constant_values=N_GRAPHS).reshape(1, N_PAD)
    b1 = b1.reshape(1, D_HID)
    b2 = b2.reshape(1, D_HID)
    b3 = b3.reshape(1, D_HID)
    lin_b2 = lin_b.reshape(1, D_OUT)

    degp = _sc_degree(dst2)
    d0, d1 = degp[0], degp[1]

    y1 = _tc_layer1(x, W1, d0, d1)
    a = _sc_message(y1, src2, dst2)
    y2 = _tc_layer(a[0], a[1], y1, d0, d1, b1, W2)
    a = _sc_message(y2, src2, dst2)
    y3 = _tc_layer(a[0], a[1], y2, d0, d1, b2, W3)
    a = _sc_message(y3, src2, dst2)
    return _tc_final(a[0], a[1], y3, d0, d1, b3, batch2, lin_W, lin_b2)


# speedup vs baseline: 2.9162x; 2.9162x over previous
import functools

import jax
import jax.numpy as jnp
from jax import lax
from jax.experimental import pallas as pl
from jax.experimental.pallas import tpu as pltpu
from jax.experimental.pallas import tpu_sc as plsc

N_NODES = 10000
N_EDGES = 320000
D_HID = 128
N_GRAPHS = 128
D_OUT = 10

NC = 2
NS = 16
NW = NC * NS
ECHUNK = 128
E_PAD = 327680
EROWS = E_PAD // ECHUNK
ROWS_PER_W = EROWS // NW
N_PAD = 10240
ROWS_PER_TILE = N_PAD // NS
ROW_COPY = 128

_sc_mesh = plsc.VectorSubcoreMesh(core_axis_name="c", subcore_axis_name="s",
                                  num_cores=NC, num_subcores=NS)


def _fill(ref, val):
    r, c = ref.shape

    def body(i, _):
        for j in range(c // 16):
            ref[i, pl.ds(j * 16, 16)] = jnp.full((16,), val, jnp.float32)
        return 0

    lax.fori_loop(0, r, body, 0)


DEG_WIN = 8


@functools.partial(
    pl.kernel,
    out_type=jax.ShapeDtypeStruct((NC, N_PAD, D_HID), jnp.float32),
    mesh=_sc_mesh,
    scratch_types=[
        pltpu.VMEM((ROWS_PER_W, ECHUNK), jnp.int32),
        pltpu.VMEM((ECHUNK, D_HID), jnp.float32),
        pltpu.VMEM_SHARED((N_PAD, D_HID), jnp.float32),
        pltpu.SemaphoreType.DMA,
    ],
)
def _sc_degree(dst_hbm, out_hbm, dsts_v, ones_v, acc_sh, sem):
    c = lax.axis_index("c")
    s = lax.axis_index("s")
    wid = s * NC + c
    erow0 = wid * ROWS_PER_W

    _fill(ones_v, 0.0)
    base = s * ROWS_PER_TILE
    for k in range(ROWS_PER_TILE // ROW_COPY):
        pltpu.sync_copy(ones_v, acc_sh.at[pl.ds(base + k * ROW_COPY, ROW_COPY)])
    _fill(ones_v, 1.0)
    pltpu.sync_copy(dst_hbm.at[pl.ds(erow0, ROWS_PER_W)], dsts_v)
    plsc.subcore_barrier()

    def body(k, _):
        pltpu.async_copy(ones_v, acc_sh.at[dsts_v.at[k]], sem, add=True)

        @pl.when(k >= DEG_WIN)
        def _():
            pltpu.make_async_copy(ones_v, acc_sh.at[dsts_v.at[0]], sem).wait()

        return 0

    lax.fori_loop(0, ROWS_PER_W, body, 0)
    for _ in range(DEG_WIN):
        pltpu.make_async_copy(ones_v, acc_sh.at[dsts_v.at[0]], sem).wait()
    plsc.subcore_barrier()

    for k in range(ROWS_PER_TILE // ROW_COPY):
        r0 = base + k * ROW_COPY
        pltpu.sync_copy(acc_sh.at[pl.ds(r0, ROW_COPY)], ones_v)
        pltpu.sync_copy(ones_v, out_hbm.at[c, pl.ds(r0, ROW_COPY)])


@functools.partial(
    pl.kernel,
    out_type=jax.ShapeDtypeStruct((NC, N_PAD, D_HID), jnp.float32),
    mesh=_sc_mesh,
    scratch_types=[
        pltpu.VMEM((ROWS_PER_W // 2, ECHUNK), jnp.int32),
        pltpu.VMEM((ROWS_PER_W // 2, ECHUNK), jnp.int32),
        pltpu.VMEM((2, ECHUNK, D_HID), jnp.float32),
        pltpu.VMEM_SHARED((N_PAD, D_HID), jnp.float32),
        pltpu.SemaphoreType.DMA,
        pltpu.SemaphoreType.DMA,
    ],
)
def _sc_message(y_hbm, src_hbm, dst_hbm, out_hbm, srcs_v, dsts_v, rows_v,
                acc_sh, gsem0, gsem1):
    c = lax.axis_index("c")
    s = lax.axis_index("s")
    wid = s * NC + c
    erow0 = wid * ROWS_PER_W

    _fill(rows_v.at[0], 0.0)
    base = s * ROWS_PER_TILE
    for k in range(ROWS_PER_TILE // ROW_COPY):
        pltpu.sync_copy(rows_v.at[0], acc_sh.at[pl.ds(base + k * ROW_COPY, ROW_COPY)])
    plsc.subcore_barrier()

    half = ROWS_PER_W // 2
    for h in range(2):
        pltpu.sync_copy(src_hbm.at[pl.ds(erow0 + h * half, half)], srcs_v)
        pltpu.sync_copy(dst_hbm.at[pl.ds(erow0 + h * half, half)], dsts_v)
        pltpu.async_copy(y_hbm.at[srcs_v.at[0]], rows_v.at[0], gsem0)

        def body(i, _):
            k0 = 2 * i
            k1 = k0 + 1
            pltpu.make_async_copy(y_hbm.at[srcs_v.at[k0]], rows_v.at[0], gsem0).wait()
            pltpu.async_copy(y_hbm.at[srcs_v.at[k1]], rows_v.at[1], gsem1)
            pltpu.sync_copy(rows_v.at[0], acc_sh.at[dsts_v.at[k0]], add=True)
            pltpu.make_async_copy(y_hbm.at[srcs_v.at[k1]], rows_v.at[1], gsem1).wait()

            @pl.when(i < half // 2 - 1)
            def _():
                pltpu.async_copy(y_hbm.at[srcs_v.at[k0 + 2]], rows_v.at[0], gsem0)

            pltpu.sync_copy(rows_v.at[1], acc_sh.at[dsts_v.at[k1]], add=True)
            return 0

        lax.fori_loop(0, half // 2, body, 0)
    plsc.subcore_barrier()

    for k in range(ROWS_PER_TILE // ROW_COPY):
        r0 = base + k * ROW_COPY
        pltpu.sync_copy(acc_sh.at[pl.ds(r0, ROW_COPY)], rows_v.at[0])
        pltpu.sync_copy(rows_v.at[0], out_hbm.at[c, pl.ds(r0, ROW_COPY)])


BN = 2048


def _dinv_block(d0_ref, d1_ref):
    deg = d0_ref[:, 0:1] + d1_ref[:, 0:1] + 1.0
    return lax.rsqrt(deg)


def _tc_layer1_body(x_ref, w_ref, d0_ref, d1_ref, y_ref):
    dinv = _dinv_block(d0_ref, d1_ref)
    y_ref[...] = dinv * jnp.dot(x_ref[...], w_ref[...], preferred_element_type=jnp.float32)


def _tc_layer_body(a0_ref, a1_ref, y_ref, d0_ref, d1_ref, b_ref, w_ref, out_ref):
    dinv = _dinv_block(d0_ref, d1_ref)
    h = dinv * (a0_ref[...] + a1_ref[...] + y_ref[...]) + b_ref[...]
    h = jnp.maximum(h, 0.0)
    out_ref[...] = dinv * jnp.dot(h, w_ref[...], preferred_element_type=jnp.float32)


def _tc_final_body(a0_ref, a1_ref, y_ref, d0_ref, d1_ref, b_ref, batch_ref,
                   lw_ref, lb_ref, out_ref, pool_acc, cnt_acc):
    j = pl.program_id(0)
    dinv = _dinv_block(d0_ref, d1_ref)
    h = dinv * (a0_ref[...] + a1_ref[...] + y_ref[...]) + b_ref[...]

    bblk = batch_ref[0, :]
    onehot = (lax.broadcasted_iota(jnp.int32, (N_GRAPHS, BN), 0)
              == bblk[None, :]).astype(jnp.float32)

    @pl.when(j == 0)
    def _():
        pool_acc[...] = jnp.zeros_like(pool_acc)
        cnt_acc[...] = jnp.zeros_like(cnt_acc)

    pool_acc[...] += jnp.dot(onehot, h, preferred_element_type=jnp.float32)
    cnt_acc[...] += jnp.sum(onehot, axis=1, keepdims=True)

    @pl.when(j == pl.num_programs(0) - 1)
    def _():
        pooled = pool_acc[...] / jnp.maximum(cnt_acc[...], 1.0)
        out_ref[...] = (jnp.dot(pooled, lw_ref[...], preferred_element_type=jnp.float32)
                        + lb_ref[...])


_GRID = N_PAD // BN

_node_spec = pl.BlockSpec((BN, D_HID), lambda j: (j, 0))
_deg_spec = pl.BlockSpec((BN, D_HID), lambda j: (j, 0))
_full_spec = pl.BlockSpec((D_HID, D_HID), lambda j: (0, 0))
_bias_spec = pl.BlockSpec((1, D_HID), lambda j: (0, 0))


def _tc_layer1(x, W, d0, d1):
    return pl.pallas_call(
        _tc_layer1_body,
        grid=(_GRID,),
        in_specs=[_node_spec, _full_spec, _deg_spec, _deg_spec],
        out_specs=_node_spec,
        out_shape=jax.ShapeDtypeStruct((N_PAD, D_HID), jnp.float32),
    )(x, W, d0, d1)


def _tc_layer(a0, a1, y, d0, d1, b, W):
    return pl.pallas_call(
        _tc_layer_body,
        grid=(_GRID,),
        in_specs=[_node_spec, _node_spec, _node_spec, _deg_spec, _deg_spec,
                  _bias_spec, _full_spec],
        out_specs=_node_spec,
        out_shape=jax.ShapeDtypeStruct((N_PAD, D_HID), jnp.float32),
    )(a0, a1, y, d0, d1, b, W)


def _tc_final(a0, a1, y, d0, d1, b, batch2, lin_W, lin_b):
    return pl.pallas_call(
        _tc_final_body,
        grid=(_GRID,),
        in_specs=[_node_spec, _node_spec, _node_spec, _deg_spec, _deg_spec,
                  _bias_spec,
                  pl.BlockSpec((1, BN), lambda j: (0, j)),
                  pl.BlockSpec((D_HID, D_OUT), lambda j: (0, 0)),
                  pl.BlockSpec((1, D_OUT), lambda j: (0, 0))],
        out_specs=pl.BlockSpec((N_GRAPHS, D_OUT), lambda j: (0, 0)),
        out_shape=jax.ShapeDtypeStruct((N_GRAPHS, D_OUT), jnp.float32),
        scratch_shapes=[pltpu.VMEM((N_GRAPHS, D_HID), jnp.float32),
                        pltpu.VMEM((N_GRAPHS, 1), jnp.float32)],
    )(a0, a1, y, d0, d1, b, batch2, lin_W, lin_b)


def kernel(x, edge_index, batch, W1, b1, W2, b2, W3, b3, lin_W, lin_b):
    x = jnp.pad(x.astype(jnp.float32), ((0, N_PAD - N_NODES), (0, 0)))
    src = edge_index[0].astype(jnp.int32)
    dst = edge_index[1].astype(jnp.int32)
    npad_e = E_PAD - N_EDGES
    pad_iota = jnp.arange(npad_e, dtype=jnp.int32)
    src2 = jnp.concatenate([src, pad_iota % N_NODES]).reshape(EROWS, ECHUNK)
    dst2 = jnp.concatenate([dst, N_NODES + pad_iota % (N_PAD - N_NODES)]
                           ).reshape(EROWS, ECHUNK)
    batch2 = jnp.pad(batch.astype(jnp.int32), (0, N_PAD - N_NODES),
                     constant_values=N_GRAPHS).reshape(1, N_PAD)
    b1 = b1.reshape(1, D_HID)
    b2 = b2.reshape(1, D_HID)
    b3 = b3.reshape(1, D_HID)
    lin_b2 = lin_b.reshape(1, D_OUT)

    degp = _sc_degree(dst2)
    d0, d1 = degp[0], degp[1]

    y1 = _tc_layer1(x, W1, d0, d1)
    a = _sc_message(y1, src2, dst2)
    y2 = _tc_layer(a[0], a[1], y1, d0, d1, b1, W2)
    a = _sc_message(y2, src2, dst2)
    y3 = _tc_layer(a[0], a[1], y2, d0, d1, b2, W3)
    a = _sc_message(y3, src2, dst2)
    return _tc_final(a[0], a[1], y3, d0, d1, b3, batch2, lin_W, lin_b2)


# speedup vs baseline: 2.9187x; 1.0009x over previous
import functools

import jax
import jax.numpy as jnp
from jax import lax
from jax.experimental import pallas as pl
from jax.experimental.pallas import tpu as pltpu
from jax.experimental.pallas import tpu_sc as plsc

N_NODES = 10000
N_EDGES = 320000
D_HID = 128
N_GRAPHS = 128
D_OUT = 10

NC = 2
NS = 16
NW = NC * NS
ECHUNK = 128
E_PAD = 327680
EROWS = E_PAD // ECHUNK
ROWS_PER_W = EROWS // NW
N_PAD = 10240
ROWS_PER_TILE = N_PAD // NS
ROW_COPY = 128

_sc_mesh = plsc.VectorSubcoreMesh(core_axis_name="c", subcore_axis_name="s",
                                  num_cores=NC, num_subcores=NS)


def _fill(ref, val):
    r, c = ref.shape

    def body(i, _):
        for j in range(c // 16):
            ref[i, pl.ds(j * 16, 16)] = jnp.full((16,), val, jnp.float32)
        return 0

    lax.fori_loop(0, r, body, 0)


DEG_WIN = 8


@functools.partial(
    pl.kernel,
    out_type=jax.ShapeDtypeStruct((NC, N_PAD, D_HID), jnp.float32),
    mesh=_sc_mesh,
    scratch_types=[
        pltpu.VMEM((ROWS_PER_W, ECHUNK), jnp.int32),
        pltpu.VMEM((ECHUNK, D_HID), jnp.float32),
        pltpu.VMEM_SHARED((N_PAD, D_HID), jnp.float32),
        pltpu.SemaphoreType.DMA,
    ],
)
def _sc_degree(dst_hbm, out_hbm, dsts_v, ones_v, acc_sh, sem):
    c = lax.axis_index("c")
    s = lax.axis_index("s")
    wid = s * NC + c
    erow0 = wid * ROWS_PER_W

    _fill(ones_v, 0.0)
    base = s * ROWS_PER_TILE
    for k in range(ROWS_PER_TILE // ROW_COPY):
        pltpu.async_copy(ones_v, acc_sh.at[pl.ds(base + k * ROW_COPY, ROW_COPY)], sem)
    pltpu.sync_copy(dst_hbm.at[pl.ds(erow0, ROWS_PER_W)], dsts_v)
    for k in range(ROWS_PER_TILE // ROW_COPY):
        pltpu.make_async_copy(ones_v, acc_sh.at[pl.ds(base, ROW_COPY)], sem).wait()
    _fill(ones_v, 1.0)
    plsc.subcore_barrier()

    def body(k, _):
        pltpu.async_copy(ones_v, acc_sh.at[dsts_v.at[k]], sem, add=True)

        @pl.when(k >= DEG_WIN)
        def _():
            pltpu.make_async_copy(ones_v, acc_sh.at[dsts_v.at[0]], sem).wait()

        return 0

    lax.fori_loop(0, ROWS_PER_W, body, 0)
    for _ in range(DEG_WIN):
        pltpu.make_async_copy(ones_v, acc_sh.at[dsts_v.at[0]], sem).wait()
    plsc.subcore_barrier()

    pltpu.sync_copy(acc_sh.at[pl.ds(base, ROWS_PER_TILE)],
                    out_hbm.at[c, pl.ds(base, ROWS_PER_TILE)])


@functools.partial(
    pl.kernel,
    out_type=jax.ShapeDtypeStruct((NC, N_PAD, D_HID), jnp.float32),
    mesh=_sc_mesh,
    scratch_types=[
        pltpu.VMEM((ROWS_PER_W // 2, ECHUNK), jnp.int32),
        pltpu.VMEM((ROWS_PER_W // 2, ECHUNK), jnp.int32),
        pltpu.VMEM((2, ECHUNK, D_HID), jnp.float32),
        pltpu.VMEM_SHARED((N_PAD, D_HID), jnp.float32),
        pltpu.SemaphoreType.DMA,
        pltpu.SemaphoreType.DMA,
    ],
)
def _sc_message(y_hbm, src_hbm, dst_hbm, out_hbm, srcs_v, dsts_v, rows_v,
                acc_sh, gsem0, gsem1):
    c = lax.axis_index("c")
    s = lax.axis_index("s")
    wid = s * NC + c
    erow0 = wid * ROWS_PER_W

    _fill(rows_v.at[0], 0.0)
    base = s * ROWS_PER_TILE
    for k in range(ROWS_PER_TILE // ROW_COPY):
        pltpu.async_copy(rows_v.at[0], acc_sh.at[pl.ds(base + k * ROW_COPY, ROW_COPY)],
                         gsem0)
    for k in range(ROWS_PER_TILE // ROW_COPY):
        pltpu.make_async_copy(rows_v.at[0], acc_sh.at[pl.ds(base, ROW_COPY)],
                              gsem0).wait()
    plsc.subcore_barrier()

    half = ROWS_PER_W // 2
    for h in range(2):
        pltpu.sync_copy(src_hbm.at[pl.ds(erow0 + h * half, half)], srcs_v)
        pltpu.sync_copy(dst_hbm.at[pl.ds(erow0 + h * half, half)], dsts_v)
        pltpu.async_copy(y_hbm.at[srcs_v.at[0]], rows_v.at[0], gsem0)

        def body(i, _):
            k0 = 2 * i
            k1 = k0 + 1
            pltpu.make_async_copy(y_hbm.at[srcs_v.at[k0]], rows_v.at[0], gsem0).wait()
            pltpu.async_copy(y_hbm.at[srcs_v.at[k1]], rows_v.at[1], gsem1)
            pltpu.sync_copy(rows_v.at[0], acc_sh.at[dsts_v.at[k0]], add=True)
            pltpu.make_async_copy(y_hbm.at[srcs_v.at[k1]], rows_v.at[1], gsem1).wait()

            @pl.when(i < half // 2 - 1)
            def _():
                pltpu.async_copy(y_hbm.at[srcs_v.at[k0 + 2]], rows_v.at[0], gsem0)

            pltpu.sync_copy(rows_v.at[1], acc_sh.at[dsts_v.at[k1]], add=True)
            return 0

        lax.fori_loop(0, half // 2, body, 0)
    plsc.subcore_barrier()

    pltpu.sync_copy(acc_sh.at[pl.ds(base, ROWS_PER_TILE)],
                    out_hbm.at[c, pl.ds(base, ROWS_PER_TILE)])


BN = 2048


def _dinv_block(d0_ref, d1_ref):
    deg = d0_ref[:, 0:1] + d1_ref[:, 0:1] + 1.0
    return lax.rsqrt(deg)


def _tc_layer1_body(x_ref, w_ref, d0_ref, d1_ref, y_ref):
    dinv = _dinv_block(d0_ref, d1_ref)
    y_ref[...] = dinv * jnp.dot(x_ref[...], w_ref[...], preferred_element_type=jnp.float32)


def _tc_layer_body(a0_ref, a1_ref, y_ref, d0_ref, d1_ref, b_ref, w_ref, out_ref):
    dinv = _dinv_block(d0_ref, d1_ref)
    h = dinv * (a0_ref[...] + a1_ref[...] + y_ref[...]) + b_ref[...]
    h = jnp.maximum(h, 0.0)
    out_ref[...] = dinv * jnp.dot(h, w_ref[...], preferred_element_type=jnp.float32)


def _tc_final_body(a0_ref, a1_ref, y_ref, d0_ref, d1_ref, b_ref, batch_ref,
                   lw_ref, lb_ref, out_ref, pool_acc, cnt_acc):
    j = pl.program_id(0)
    dinv = _dinv_block(d0_ref, d1_ref)
    h = dinv * (a0_ref[...] + a1_ref[...] + y_ref[...]) + b_ref[...]

    bblk = batch_ref[0, :]
    onehot = (lax.broadcasted_iota(jnp.int32, (N_GRAPHS, BN), 0)
              == bblk[None, :]).astype(jnp.float32)

    @pl.when(j == 0)
    def _():
        pool_acc[...] = jnp.zeros_like(pool_acc)
        cnt_acc[...] = jnp.zeros_like(cnt_acc)

    pool_acc[...] += jnp.dot(onehot, h, preferred_element_type=jnp.float32)
    cnt_acc[...] += jnp.sum(onehot, axis=1, keepdims=True)

    @pl.when(j == pl.num_programs(0) - 1)
    def _():
        pooled = pool_acc[...] / jnp.maximum(cnt_acc[...], 1.0)
        out_ref[...] = (jnp.dot(pooled, lw_ref[...], preferred_element_type=jnp.float32)
                        + lb_ref[...])


_GRID = N_PAD // BN

_node_spec = pl.BlockSpec((BN, D_HID), lambda j: (j, 0))
_deg_spec = pl.BlockSpec((BN, D_HID), lambda j: (j, 0))
_full_spec = pl.BlockSpec((D_HID, D_HID), lambda j: (0, 0))
_bias_spec = pl.BlockSpec((1, D_HID), lambda j: (0, 0))


def _tc_layer1(x, W, d0, d1):
    return pl.pallas_call(
        _tc_layer1_body,
        grid=(_GRID,),
        in_specs=[_node_spec, _full_spec, _deg_spec, _deg_spec],
        out_specs=_node_spec,
        out_shape=jax.ShapeDtypeStruct((N_PAD, D_HID), jnp.float32),
    )(x, W, d0, d1)


def _tc_layer(a0, a1, y, d0, d1, b, W):
    return pl.pallas_call(
        _tc_layer_body,
        grid=(_GRID,),
        in_specs=[_node_spec, _node_spec, _node_spec, _deg_spec, _deg_spec,
                  _bias_spec, _full_spec],
        out_specs=_node_spec,
        out_shape=jax.ShapeDtypeStruct((N_PAD, D_HID), jnp.float32),
    )(a0, a1, y, d0, d1, b, W)


def _tc_final(a0, a1, y, d0, d1, b, batch2, lin_W, lin_b):
    return pl.pallas_call(
        _tc_final_body,
        grid=(_GRID,),
        in_specs=[_node_spec, _node_spec, _node_spec, _deg_spec, _deg_spec,
                  _bias_spec,
                  pl.BlockSpec((1, BN), lambda j: (0, j)),
                  pl.BlockSpec((D_HID, D_OUT), lambda j: (0, 0)),
                  pl.BlockSpec((1, D_OUT), lambda j: (0, 0))],
        out_specs=pl.BlockSpec((N_GRAPHS, D_OUT), lambda j: (0, 0)),
        out_shape=jax.ShapeDtypeStruct((N_GRAPHS, D_OUT), jnp.float32),
        scratch_shapes=[pltpu.VMEM((N_GRAPHS, D_HID), jnp.float32),
                        pltpu.VMEM((N_GRAPHS, 1), jnp.float32)],
    )(a0, a1, y, d0, d1, b, batch2, lin_W, lin_b)


def kernel(x, edge_index, batch, W1, b1, W2, b2, W3, b3, lin_W, lin_b):
    x = jnp.pad(x.astype(jnp.float32), ((0, N_PAD - N_NODES), (0, 0)))
    src = edge_index[0].astype(jnp.int32)
    dst = edge_index[1].astype(jnp.int32)
    npad_e = E_PAD - N_EDGES
    pad_iota = jnp.arange(npad_e, dtype=jnp.int32)
    src2 = jnp.concatenate([src, pad_iota % N_NODES]).reshape(EROWS, ECHUNK)
    dst2 = jnp.concatenate([dst, N_NODES + pad_iota % (N_PAD - N_NODES)]
                           ).reshape(EROWS, ECHUNK)
    batch2 = jnp.pad(batch.astype(jnp.int32), (0, N_PAD - N_NODES),
                     constant_values=N_GRAPHS).reshape(1, N_PAD)
    b1 = b1.reshape(1, D_HID)
    b2 = b2.reshape(1, D_HID)
    b3 = b3.reshape(1, D_HID)
    lin_b2 = lin_b.reshape(1, D_OUT)

    degp = _sc_degree(dst2)
    d0, d1 = degp[0], degp[1]

    y1 = _tc_layer1(x, W1, d0, d1)
    a = _sc_message(y1, src2, dst2)
    y2 = _tc_layer(a[0], a[1], y1, d0, d1, b1, W2)
    a = _sc_message(y2, src2, dst2)
    y3 = _tc_layer(a[0], a[1], y2, d0, d1, b2, W3)
    a = _sc_message(y3, src2, dst2)
    return _tc_final(a[0], a[1], y3, d0, d1, b3, batch2, lin_W, lin_b2)
